# Initial kernel scaffold; baseline (speedup 1.0000x reference)
import jax
import jax.numpy as jnp
from jax.experimental import pallas as pl


def kernel(x, lap_indices, lap_values, W, b):
    raise NotImplementedError("write your pallas kernel here")


# speedup vs baseline: 6.6108x; 6.6108x over previous
import jax
import jax.numpy as jnp
from jax import lax
from jax.experimental import pallas as pl
from jax.experimental.pallas import tpu as pltpu
from jax.experimental.pallas import tpu_sc as plsc

N = 10000
D = 128
E = 320000
NC = 2
NS = 16
NW = NC * NS
EPW = E // NW
CH = 80
NCHUNK = EPW // CH
ACC_ROWS = 10240
RPT = ACC_ROWS // NS
ZROWS = 128
MB = 1000


def _mm_body(x_ref, w_ref, o_ref):
    o_ref[...] = jnp.dot(x_ref[...], w_ref[...],
                         preferred_element_type=jnp.float32)


def _matmul(x, W):
    return pl.pallas_call(
        _mm_body,
        grid=(N // MB,),
        in_specs=[pl.BlockSpec((MB, D), lambda i: (i, 0)),
                  pl.BlockSpec((D, D), lambda i: (0, 0))],
        out_specs=pl.BlockSpec((MB, D), lambda i: (i, 0)),
        out_shape=jax.ShapeDtypeStruct((N, D), jnp.float32),
    )(x, W)


def _comb_body(p_ref, b_ref, o_ref):
    o_ref[...] = jnp.maximum(p_ref[0] + p_ref[1] + b_ref[...], 0.0)


def _combine(partials, b2):
    return pl.pallas_call(
        _comb_body,
        grid=(N // MB,),
        in_specs=[pl.BlockSpec((NC, MB, D), lambda i: (0, i, 0)),
                  pl.BlockSpec((1, D), lambda i: (0, 0))],
        out_specs=pl.BlockSpec((MB, D), lambda i: (i, 0)),
        out_shape=jax.ShapeDtypeStruct((N, D), jnp.float32),
    )(partials, b2)


def _sc_body(t_hbm, src_hbm, dst_hbm, val_hbm, out_hbm,
             src_v, dst_v, val_v, rows_v, acc_sh, gsem, vsem):
    cid = lax.axis_index("c")
    sid = lax.axis_index("s")
    wid = sid * NC + cid

    def zrow(i, c):
        for k in range(D // 16):
            rows_v[i, pl.ds(k * 16, 16)] = jnp.zeros((16,), jnp.float32)
        return c
    lax.fori_loop(0, CH, zrow, 0)
    for m in range(RPT // CH):
        pltpu.sync_copy(rows_v,
                        acc_sh.at[pl.ds(sid * RPT + m * CH, CH)])
    plsc.subcore_barrier()

    pltpu.sync_copy(src_hbm.at[wid], src_v)
    pltpu.sync_copy(dst_hbm.at[wid], dst_v)

    def chunk(j, c):
        vcp = pltpu.async_copy(val_hbm.at[wid, j], val_v, vsem)
        gcp = pltpu.async_copy(t_hbm.at[src_v.at[j]], rows_v, gsem)
        vcp.wait()
        gcp.wait()

        def group(g, c2):
            base = g * 16
            v16 = val_v[pl.ds(base, 16)]
            for l in range(16):
                vv = v16[l]
                for k in range(D // 16):
                    sl = pl.ds(k * 16, 16)
                    rows_v[base + l, sl] = rows_v[base + l, sl] * vv
            return c2
        lax.fori_loop(0, CH // 16, group, 0)
        pltpu.sync_copy(rows_v, acc_sh.at[dst_v.at[j]], add=True)
        return c
    lax.fori_loop(0, NCHUNK, chunk, 0)

    plsc.subcore_barrier()
    for m in range(RPT // ZROWS):
        r0 = sid * RPT + m * ZROWS
        pltpu.sync_copy(acc_sh.at[pl.ds(r0, ZROWS)],
                        out_hbm.at[cid, pl.ds(r0, ZROWS)])


_sc_call = pl.kernel(
    _sc_body,
    out_type=jax.ShapeDtypeStruct((NC, ACC_ROWS, D), jnp.float32),
    mesh=plsc.VectorSubcoreMesh(core_axis_name="c", subcore_axis_name="s"),
    scratch_types=[
        pltpu.VMEM((NCHUNK, CH), jnp.int32),
        pltpu.VMEM((NCHUNK, CH), jnp.int32),
        pltpu.VMEM((CH,), jnp.float32),
        pltpu.VMEM((CH, D), jnp.float32),
        pltpu.VMEM_SHARED((ACC_ROWS, D), jnp.float32),
        pltpu.SemaphoreType.DMA,
        pltpu.SemaphoreType.DMA,
    ],
)


def kernel(x, lap_indices, lap_values, W, b):
    T = _matmul(x, W)
    dst = lap_indices[0].reshape(NW, NCHUNK, CH)
    src = lap_indices[1].reshape(NW, NCHUNK, CH)
    val = lap_values.reshape(NW, NCHUNK, CH)
    partials = _sc_call(T, src, dst, val)
    return _combine(partials, b.reshape(1, D))


# speedup vs baseline: 11.2621x; 1.7036x over previous
import jax
import jax.numpy as jnp
from jax import lax
from jax.experimental import pallas as pl
from jax.experimental.pallas import tpu as pltpu
from jax.experimental.pallas import tpu_sc as plsc

N = 10000
D = 128
E = 320000
NC = 2
NS = 16
NW = NC * NS
EPW = E // NW
CH = 80
NCHUNK = EPW // CH
ACC_ROWS = 10240
RPT = ACC_ROWS // NS
MB = 1000


def _fuse_body(p_ref, w_ref, b_ref, o_ref):
    s = p_ref[0] + p_ref[1]
    o_ref[...] = jnp.maximum(
        jnp.dot(s, w_ref[...], preferred_element_type=jnp.float32)
        + b_ref[...], 0.0)


def _fuse(partials, W, b2):
    return pl.pallas_call(
        _fuse_body,
        grid=(N // MB,),
        in_specs=[pl.BlockSpec((NC, MB, D), lambda i: (0, i, 0)),
                  pl.BlockSpec((D, D), lambda i: (0, 0)),
                  pl.BlockSpec((1, D), lambda i: (0, 0))],
        out_specs=pl.BlockSpec((MB, D), lambda i: (i, 0)),
        out_shape=jax.ShapeDtypeStruct((N, D), jnp.float32),
    )(partials, W, b2)


def _scale(rows_ref, val_ref):
    def group(g, c):
        base = g * 16
        v16 = val_ref[pl.ds(base, 16)]
        for l in range(16):
            vv = v16[l]
            for k in range(D // 16):
                sl = pl.ds(k * 16, 16)
                rows_ref[base + l, sl] = rows_ref[base + l, sl] * vv
        return c
    lax.fori_loop(0, CH // 16, group, 0)


def _sc_body(x_hbm, src_hbm, dst_hbm, val_hbm, out_hbm,
             src_v, dst_a, dst_b, val_a, val_b, rows_a, rows_b, acc_sh,
             ga, gb, va, vb, da, db):
    cid = lax.axis_index("c")
    sid = lax.axis_index("s")
    wid = sid * NC + cid

    def zrow(i, c):
        for k in range(D // 16):
            rows_a[i, pl.ds(k * 16, 16)] = jnp.zeros((16,), jnp.float32)
        return c
    lax.fori_loop(0, CH, zrow, 0)
    for m in range(RPT // CH):
        pltpu.sync_copy(rows_a, acc_sh.at[pl.ds(sid * RPT + m * CH, CH)])
    plsc.subcore_barrier()

    pltpu.sync_copy(src_hbm.at[wid], src_v)

    ebase = wid * EPW
    pltpu.async_copy(val_hbm.at[pl.ds(ebase, CH)], val_a, va)
    pltpu.async_copy(dst_hbm.at[pl.ds(ebase, CH)], dst_a, da)
    pltpu.async_copy(x_hbm.at[src_v.at[0]], rows_a, ga)
    pltpu.async_copy(val_hbm.at[pl.ds(ebase + CH, CH)], val_b, vb)
    pltpu.async_copy(dst_hbm.at[pl.ds(ebase + CH, CH)], dst_b, db)
    pltpu.async_copy(x_hbm.at[src_v.at[1]], rows_b, gb)

    def half(j, rows_ref, val_ref, dst_ref, gsem, vsem, dsem):
        pltpu.make_async_copy(x_hbm.at[src_v.at[j]], rows_ref, gsem).wait()
        pltpu.make_async_copy(val_hbm.at[pl.ds(ebase, CH)], val_ref, vsem).wait()
        pltpu.make_async_copy(dst_hbm.at[pl.ds(ebase, CH)], dst_ref, dsem).wait()
        _scale(rows_ref, val_ref)
        pltpu.sync_copy(rows_ref, acc_sh.at[dst_ref], add=True)

        @pl.when(j + 2 < NCHUNK)
        def _():
            off = ebase + (j + 2) * CH
            pltpu.async_copy(val_hbm.at[pl.ds(off, CH)], val_ref, vsem)
            pltpu.async_copy(dst_hbm.at[pl.ds(off, CH)], dst_ref, dsem)
            pltpu.async_copy(x_hbm.at[src_v.at[j + 2]], rows_ref, gsem)

    def pair(jj, c):
        j0 = 2 * jj
        half(j0, rows_a, val_a, dst_a, ga, va, da)
        half(j0 + 1, rows_b, val_b, dst_b, gb, vb, db)
        return c
    lax.fori_loop(0, NCHUNK // 2, pair, 0)
    pltpu.make_async_copy(x_hbm.at[src_v.at[NCHUNK - 1]], rows_a, ga).wait()
    pltpu.make_async_copy(val_hbm.at[pl.ds(ebase, CH)], val_a, va).wait()
    pltpu.make_async_copy(dst_hbm.at[pl.ds(ebase, CH)], dst_a, da).wait()
    _scale(rows_a, val_a)
    pltpu.sync_copy(rows_a, acc_sh.at[dst_a], add=True)

    plsc.subcore_barrier()
    for m in range(RPT // CH):
        r0 = sid * RPT + m * CH
        pltpu.sync_copy(acc_sh.at[pl.ds(r0, CH)],
                        out_hbm.at[cid, pl.ds(r0, CH)])


_sc_call = pl.kernel(
    _sc_body,
    out_type=jax.ShapeDtypeStruct((NC, ACC_ROWS, D), jnp.float32),
    mesh=plsc.VectorSubcoreMesh(core_axis_name="c", subcore_axis_name="s"),
    scratch_types=[
        pltpu.VMEM((NCHUNK, CH), jnp.int32),
        pltpu.VMEM((CH,), jnp.int32),
        pltpu.VMEM((CH,), jnp.int32),
        pltpu.VMEM((CH,), jnp.float32),
        pltpu.VMEM((CH,), jnp.float32),
        pltpu.VMEM((CH, D), jnp.float32),
        pltpu.VMEM((CH, D), jnp.float32),
        pltpu.VMEM_SHARED((ACC_ROWS, D), jnp.float32),
        pltpu.SemaphoreType.DMA,
        pltpu.SemaphoreType.DMA,
        pltpu.SemaphoreType.DMA,
        pltpu.SemaphoreType.DMA,
        pltpu.SemaphoreType.DMA,
        pltpu.SemaphoreType.DMA,
    ],
)


def kernel(x, lap_indices, lap_values, W, b):
    dst = lap_indices[0]
    src = lap_indices[1].reshape(NW, NCHUNK, CH)
    partials = _sc_call(x, src, dst, lap_values)
    return _fuse(partials, W, b.reshape(1, D))


# speedup vs baseline: 12.6041x; 1.1192x over previous
import jax
import jax.numpy as jnp
from jax import lax
from jax.experimental import pallas as pl
from jax.experimental.pallas import tpu as pltpu
from jax.experimental.pallas import tpu_sc as plsc

N = 10000
D = 128
E = 320000
NC = 2
NS = 16
NW = NC * NS
EPW = E // NW
CH = 80
NCHUNK = EPW // CH
ACC_ROWS = 10240
RPT = ACC_ROWS // NS
MB = 1000


def _fuse_body(p_ref, w_ref, b_ref, o_ref):
    s = p_ref[0] + p_ref[1]
    o_ref[...] = jnp.maximum(
        jnp.dot(s, w_ref[...], preferred_element_type=jnp.float32)
        + b_ref[...], 0.0)


def _fuse(partials, W, b2):
    return pl.pallas_call(
        _fuse_body,
        grid=(N // MB,),
        in_specs=[pl.BlockSpec((NC, MB, D), lambda i: (0, i, 0)),
                  pl.BlockSpec((D, D), lambda i: (0, 0)),
                  pl.BlockSpec((1, D), lambda i: (0, 0))],
        out_specs=pl.BlockSpec((MB, D), lambda i: (i, 0)),
        out_shape=jax.ShapeDtypeStruct((N, D), jnp.float32),
    )(partials, W, b2)


def _scale(rows_ref, val_ref):
    def group(g, c):
        base = g * 16
        v16 = val_ref[pl.ds(base, 16)]
        for l in range(16):
            vv = v16[l]
            for k in range(D // 16):
                sl = pl.ds(k * 16, 16)
                rows_ref[base + l, sl] = rows_ref[base + l, sl] * vv
        return c
    lax.fori_loop(0, CH // 16, group, 0)


def _sc_body(x_hbm, src_hbm, dst_hbm, val_hbm, out_hbm,
             src_v, dst_a, dst_b, dst_c, val_a, val_b, val_c,
             rows_a, rows_b, rows_c, acc_sh,
             ga, gb, gc, ha, hb, hc, ka, kb, kc, sa, sb, sc):
    cid = lax.axis_index("c")
    sid = lax.axis_index("s")
    wid = sid * NC + cid
    ebase = wid * EPW

    scp = pltpu.async_copy(src_hbm.at[wid], src_v, ga)

    def zrow(i, c):
        for k in range(D // 16):
            rows_a[i, pl.ds(k * 16, 16)] = jnp.zeros((16,), jnp.float32)
        return c
    lax.fori_loop(0, CH, zrow, 0)
    for m in range(RPT // CH):
        pltpu.sync_copy(rows_a, acc_sh.at[pl.ds(sid * RPT + m * CH, CH)])
    scp.wait()
    plsc.subcore_barrier()

    def issue3(j, rows_ref, val_ref, dst_ref, gsem, hsem, ksem):
        off = ebase + j * CH
        pltpu.async_copy(val_hbm.at[pl.ds(off, CH)], val_ref, hsem)
        pltpu.async_copy(dst_hbm.at[pl.ds(off, CH)], dst_ref, ksem)
        pltpu.async_copy(x_hbm.at[src_v.at[j]], rows_ref, gsem)

    def wait3(rows_ref, val_ref, dst_ref, gsem, hsem, ksem):
        pltpu.make_async_copy(val_hbm.at[pl.ds(ebase, CH)], val_ref, hsem).wait()
        pltpu.make_async_copy(dst_hbm.at[pl.ds(ebase, CH)], dst_ref, ksem).wait()
        pltpu.make_async_copy(x_hbm.at[src_v.at[0]], rows_ref, gsem).wait()

    def drain_scatter(rows_ref, dst_ref, ssem):
        pltpu.make_async_copy(rows_ref, acc_sh.at[dst_ref], ssem).wait()

    bufs = [
        (rows_a, val_a, dst_a, ga, ha, ka, sa),
        (rows_b, val_b, dst_b, gb, hb, kb, sb),
        (rows_c, val_c, dst_c, gc, hc, kc, sc),
    ]

    def half_async(j, cur, nxt, first_prefetch=False):
        r, v, d, g, h, k, ss = bufs[cur]
        rn, vn, dn, gn, hn, kn, ssn = bufs[nxt]
        wait3(r, v, d, g, h, k)
        _scale(r, v)

        def prefetch():
            if not first_prefetch:
                drain_scatter(rn, dn, ssn)
            issue3(j + 2, rn, vn, dn, gn, hn, kn)
        if isinstance(j, int) and j + 2 < NCHUNK:
            prefetch()
        elif not isinstance(j, int):
            pl.when(j + 2 < NCHUNK)(prefetch)
        pltpu.async_copy(r, acc_sh.at[d], ss, add=True)

    issue3(0, rows_a, val_a, dst_a, ga, ha, ka)
    issue3(1, rows_b, val_b, dst_b, gb, hb, kb)
    half_async(0, 0, 2, first_prefetch=True)

    def group3(g, c):
        base = 3 * g
        half_async(base + 1, 1, 0)
        half_async(base + 2, 2, 1)
        half_async(base + 3, 0, 2)
        return c
    lax.fori_loop(0, (NCHUNK - 2) // 3, group3, 0)

    r, v, d, g, h, k, ss = bufs[1]
    wait3(r, v, d, g, h, k)
    _scale(r, v)
    pltpu.sync_copy(r, acc_sh.at[d], add=True)
    drain_scatter(rows_a, dst_a, sa)
    drain_scatter(rows_c, dst_c, sc)

    plsc.subcore_barrier()
    for m in range(RPT // CH):
        r0 = sid * RPT + m * CH
        pltpu.sync_copy(acc_sh.at[pl.ds(r0, CH)],
                        out_hbm.at[cid, pl.ds(r0, CH)])


_sc_call = pl.kernel(
    _sc_body,
    out_type=jax.ShapeDtypeStruct((NC, ACC_ROWS, D), jnp.float32),
    mesh=plsc.VectorSubcoreMesh(core_axis_name="c", subcore_axis_name="s"),
    scratch_types=[
        pltpu.VMEM((NCHUNK, CH), jnp.int32),
        pltpu.VMEM((CH,), jnp.int32),
        pltpu.VMEM((CH,), jnp.int32),
        pltpu.VMEM((CH,), jnp.int32),
        pltpu.VMEM((CH,), jnp.float32),
        pltpu.VMEM((CH,), jnp.float32),
        pltpu.VMEM((CH,), jnp.float32),
        pltpu.VMEM((CH, D), jnp.float32),
        pltpu.VMEM((CH, D), jnp.float32),
        pltpu.VMEM((CH, D), jnp.float32),
        pltpu.VMEM_SHARED((ACC_ROWS, D), jnp.float32),
    ] + [pltpu.SemaphoreType.DMA] * 12,
)


def kernel(x, lap_indices, lap_values, W, b):
    dst = lap_indices[0]
    src = lap_indices[1].reshape(NW, NCHUNK, CH)
    partials = _sc_call(x, src, dst, lap_values)
    return _fuse(partials, W, b.reshape(1, D))
